# baseline (speedup 1.0000x reference)
import functools

import jax
import jax.numpy as jnp
from jax import lax
from jax.experimental import pallas as pl
from jax.experimental.pallas import tpu as pltpu
from jax.experimental.pallas import tpu_sc as plsc

L = 16
CHUNK = 128


@functools.lru_cache(maxsize=None)
def _build(total, per_tile, n_chunks, F, nc, ns):
    mesh = plsc.VectorSubcoreMesh(core_axis_name="c", subcore_axis_name="s")
    pv = CHUNK // L

    @functools.partial(
        pl.kernel,
        mesh=mesh,
        compiler_params=pltpu.CompilerParams(
            needs_layout_passes=False, use_tc_tiling_on_sc=False),
        out_type=jax.ShapeDtypeStruct((total,), jnp.float32),
        scratch_types=[
            pltpu.VMEM((CHUNK,), jnp.int32),
            pltpu.VMEM((CHUNK,), jnp.int32),
            pltpu.VMEM((CHUNK,), jnp.int32),
            pltpu.VMEM((CHUNK, F), jnp.float32),
            pltpu.VMEM((CHUNK, F), jnp.float32),
            pltpu.VMEM((2 * F,), jnp.float32),
            pltpu.VMEM((per_tile,), jnp.float32),
            pltpu.SemaphoreType.DMA,
            pltpu.SemaphoreType.DMA,
        ],
    )
    def k(table, rtab, hids, tids, rids, out,
          hidx, tidx, relv, hbuf, tbuf, rbuf, outbuf, semh, semt):
        wid = lax.axis_index("s") * nc + lax.axis_index("c")
        base = wid * per_tile
        pltpu.sync_copy(rtab, rbuf)
        iota = lax.iota(jnp.int32, L)
        rows = [iota + p * L for p in range(pv)]
        for c in range(n_chunks):
            off = base + c * CHUNK
            pltpu.sync_copy(hids.at[pl.ds(off, CHUNK)], hidx)
            pltpu.sync_copy(tids.at[pl.ds(off, CHUNK)], tidx)
            pltpu.sync_copy(rids.at[pl.ds(off, CHUNK)], relv)
            ch = pltpu.async_copy(table.at[hidx], hbuf, semh)
            ct = pltpu.async_copy(table.at[tidx], tbuf, semt)
            ch.wait()
            ct.wait()
            rels = [relv[pl.ds(p * L, L)] * F for p in range(pv)]

            def body(f, accs):
                fs = jnp.broadcast_to(f.astype(jnp.int32), (L,))
                new = []
                for p in range(pv):
                    hv = plsc.load_gather(hbuf, [rows[p], fs])
                    tv = plsc.load_gather(tbuf, [rows[p], fs])
                    rv = plsc.load_gather(rbuf, [rels[p] + fs])
                    d = hv + rv - tv
                    new.append(accs[p] + d * d)
                return tuple(new)

            accs = lax.fori_loop(
                0, F, body,
                tuple(jnp.zeros((L,), jnp.float32) for _ in range(pv)))
            for p in range(pv):
                outbuf[pl.ds(c * CHUNK + p * L, L)] = -accs[p]
        pltpu.sync_copy(outbuf, out.at[pl.ds(base, per_tile)])

    return k


def kernel(itemEmbedding, r_table, head_ids, tail_ids, relation_ids):
    B, K = head_ids.shape
    total = B * K
    F = itemEmbedding.shape[1]
    info = plsc.get_sparse_core_info()
    nc, ns = info.num_cores, info.num_subcores
    per_tile = total // (nc * ns)
    n_chunks = per_tile // CHUNK
    k = _build(total, per_tile, n_chunks, F, nc, ns)
    out = k(itemEmbedding, r_table.reshape(-1),
            head_ids.reshape(-1).astype(jnp.int32),
            tail_ids.reshape(-1).astype(jnp.int32),
            relation_ids.reshape(-1).astype(jnp.int32))
    return out.reshape(B, K)


# speedup vs baseline: 1.0318x; 1.0318x over previous
import functools

import jax
import jax.numpy as jnp
from jax import lax
from jax.experimental import pallas as pl
from jax.experimental.pallas import tpu as pltpu
from jax.experimental.pallas import tpu_sc as plsc

L = 16
CHUNK = 128
NBUF = 3


@functools.lru_cache(maxsize=None)
def _build(total, per_tile, n_chunks, F, nc, ns):
    mesh = plsc.VectorSubcoreMesh(core_axis_name="c", subcore_axis_name="s")
    pv = CHUNK // L
    W = 2 * F

    @functools.partial(
        pl.kernel,
        mesh=mesh,
        compiler_params=pltpu.CompilerParams(
            needs_layout_passes=False, use_tc_tiling_on_sc=True),
        out_type=jax.ShapeDtypeStruct((total,), jnp.float32),
        scratch_types=[
            pltpu.VMEM((n_chunks, CHUNK), jnp.int32),
            pltpu.VMEM((n_chunks, CHUNK), jnp.int32),
            pltpu.VMEM((per_tile,), jnp.int32),
            pltpu.VMEM((per_tile,), jnp.int32),
            pltpu.VMEM((per_tile,), jnp.int32),
            pltpu.VMEM((NBUF, CHUNK, W), jnp.float32),
            pltpu.VMEM((NBUF, CHUNK, W), jnp.float32),
            pltpu.VMEM((2 * F,), jnp.float32),
            pltpu.VMEM((per_tile,), jnp.float32),
        ] + [pltpu.SemaphoreType.DMA] * (2 * NBUF),
    )
    def k(table, rtab, hrow, trow, hoff_hbm, toff_hbm, rel_hbm, out,
          hrows, trows, hoff, toff, relv, hbuf, tbuf, rbuf, outbuf, *sems):
        hsem = sems[:NBUF]
        tsem = sems[NBUF:]
        wid = lax.axis_index("s") * nc + lax.axis_index("c")
        base = wid * per_tile
        crow = wid * n_chunks
        pltpu.sync_copy(rtab, rbuf)
        pltpu.sync_copy(hrow.at[pl.ds(crow, n_chunks)], hrows)
        pltpu.sync_copy(trow.at[pl.ds(crow, n_chunks)], trows)
        pltpu.sync_copy(hoff_hbm.at[pl.ds(base, per_tile)], hoff)
        pltpu.sync_copy(toff_hbm.at[pl.ds(base, per_tile)], toff)
        pltpu.sync_copy(rel_hbm.at[pl.ds(base, per_tile)], relv)
        iota = lax.iota(jnp.int32, L)
        rows = [iota + p * L for p in range(pv)]

        def fire(c):
            b = c % NBUF
            ch = pltpu.async_copy(table.at[hrows.at[c]], hbuf.at[b], hsem[b])
            ct = pltpu.async_copy(table.at[trows.at[c]], tbuf.at[b], tsem[b])
            return ch, ct

        pending = [fire(c) for c in range(min(NBUF - 1, n_chunks))]
        for c in range(n_chunks):
            if c + NBUF - 1 < n_chunks:
                pending.append(fire(c + NBUF - 1))
            ch, ct = pending.pop(0)
            ch.wait()
            ct.wait()
            b = c % NBUF
            hb = hbuf.at[b]
            tb = tbuf.at[b]
            rels = [relv[pl.ds(c * CHUNK + p * L, L)] * F for p in range(pv)]
            hoffs = [hoff[pl.ds(c * CHUNK + p * L, L)] for p in range(pv)]
            toffs = [toff[pl.ds(c * CHUNK + p * L, L)] for p in range(pv)]

            def body(f, accs):
                fs = jnp.broadcast_to(f.astype(jnp.int32), (L,))
                new = []
                for p in range(pv):
                    hv = plsc.load_gather(hb, [rows[p], hoffs[p] + fs])
                    tv = plsc.load_gather(tb, [rows[p], toffs[p] + fs])
                    rv = plsc.load_gather(rbuf, [rels[p] + fs])
                    d = hv + rv - tv
                    new.append(accs[p] + d * d)
                return tuple(new)

            accs = lax.fori_loop(
                0, F, body,
                tuple(jnp.zeros((L,), jnp.float32) for _ in range(pv)))
            for p in range(pv):
                outbuf[pl.ds(c * CHUNK + p * L, L)] = -accs[p]
        pltpu.sync_copy(outbuf, out.at[pl.ds(base, per_tile)])

    return k


def kernel(itemEmbedding, r_table, head_ids, tail_ids, relation_ids):
    B, K = head_ids.shape
    total = B * K
    F = itemEmbedding.shape[1]
    info = plsc.get_sparse_core_info()
    nc, ns = info.num_cores, info.num_subcores
    per_tile = total // (nc * ns)
    n_chunks = per_tile // CHUNK
    k = _build(total, per_tile, n_chunks, F, nc, ns)
    hids = head_ids.reshape(-1).astype(jnp.int32)
    tids = tail_ids.reshape(-1).astype(jnp.int32)
    out = k(itemEmbedding.reshape(itemEmbedding.shape[0] // 2, 2 * F),
            r_table.reshape(-1),
            (hids >> 1).reshape(total // CHUNK, CHUNK),
            (tids >> 1).reshape(total // CHUNK, CHUNK),
            (hids & 1) * F,
            (tids & 1) * F,
            relation_ids.reshape(-1).astype(jnp.int32))
    return out.reshape(B, K)


# speedup vs baseline: 1.0457x; 1.0134x over previous
import functools

import jax
import jax.numpy as jnp
from jax import lax
from jax.experimental import pallas as pl
from jax.experimental.pallas import tpu as pltpu
from jax.experimental.pallas import tpu_sc as plsc

L = 16
CHUNK = 128


@functools.lru_cache(maxsize=None)
def _build(total, per_tile, n_chunks, F, nc, ns):
    mesh = plsc.VectorSubcoreMesh(core_axis_name="c", subcore_axis_name="s")
    pv = CHUNK // L

    @functools.partial(
        pl.kernel,
        mesh=mesh,
        compiler_params=pltpu.CompilerParams(
            needs_layout_passes=False, use_tc_tiling_on_sc=False),
        out_type=jax.ShapeDtypeStruct((total,), jnp.float32),
        scratch_types=[
            pltpu.VMEM((n_chunks, CHUNK), jnp.int32),
            pltpu.VMEM((n_chunks, CHUNK), jnp.int32),
            pltpu.VMEM((per_tile,), jnp.int32),
            pltpu.VMEM((CHUNK, F), jnp.float32),
            pltpu.VMEM((CHUNK, F), jnp.float32),
            pltpu.VMEM((CHUNK, F), jnp.float32),
            pltpu.VMEM((CHUNK, F), jnp.float32),
            pltpu.VMEM((2 * F,), jnp.float32),
            pltpu.VMEM((per_tile,), jnp.float32),
            pltpu.SemaphoreType.DMA,
            pltpu.SemaphoreType.DMA,
            pltpu.SemaphoreType.DMA,
            pltpu.SemaphoreType.DMA,
        ],
    )
    def k(table, rtab, hids, tids, rids, out,
          hrows, trows, relv, hb0, hb1, tb0, tb1, rbuf, outbuf,
          hs0, hs1, ts0, ts1):
        wid = lax.axis_index("s") * nc + lax.axis_index("c")
        base = wid * per_tile
        pltpu.sync_copy(rtab, rbuf)
        pltpu.sync_copy(hids.at[pl.ds(wid * n_chunks, n_chunks)], hrows)
        pltpu.sync_copy(tids.at[pl.ds(wid * n_chunks, n_chunks)], trows)
        pltpu.sync_copy(rids.at[pl.ds(base, per_tile)], relv)
        iota = lax.iota(jnp.int32, L)
        rows = [iota + p * L for p in range(pv)]

        def fire(c, hb, tb, hs, ts):
            pltpu.async_copy(table.at[hrows.at[c]], hb, hs)
            pltpu.async_copy(table.at[trows.at[c]], tb, ts)

        def drain(c, hb, tb, hs, ts):
            pltpu.make_async_copy(table.at[hrows.at[c]], hb, hs).wait()
            pltpu.make_async_copy(table.at[trows.at[c]], tb, ts).wait()

        def compute(c, hb, tb):
            rels = [relv[pl.ds(c * CHUNK + p * L, L)] * F for p in range(pv)]

            def body(f, accs):
                fs = jnp.broadcast_to(f.astype(jnp.int32), (L,))
                new = []
                for p in range(pv):
                    hv = plsc.load_gather(hb, [rows[p], fs])
                    tv = plsc.load_gather(tb, [rows[p], fs])
                    rv = plsc.load_gather(rbuf, [rels[p] + fs])
                    d = hv + rv - tv
                    new.append(accs[p] + d * d)
                return tuple(new)

            accs = lax.fori_loop(
                0, F, body,
                tuple(jnp.zeros((L,), jnp.float32) for _ in range(pv)))
            for p in range(pv):
                outbuf[pl.ds(c * CHUNK + p * L, L)] = -accs[p]

        fire(0, hb0, tb0, hs0, ts0)
        last = n_chunks - 1

        def step(i, carry):
            c = 2 * i
            fire(c + 1, hb1, tb1, hs1, ts1)
            drain(c, hb0, tb0, hs0, ts0)
            compute(c, hb0, tb0)
            fire(jnp.minimum(c + 2, last), hb0, tb0, hs0, ts0)
            drain(c + 1, hb1, tb1, hs1, ts1)
            compute(c + 1, hb1, tb1)
            return carry

        lax.fori_loop(0, n_chunks // 2, step, 0)
        drain(last, hb0, tb0, hs0, ts0)
        pltpu.sync_copy(outbuf, out.at[pl.ds(base, per_tile)])

    return k


def kernel(itemEmbedding, r_table, head_ids, tail_ids, relation_ids):
    B, K = head_ids.shape
    total = B * K
    F = itemEmbedding.shape[1]
    info = plsc.get_sparse_core_info()
    nc, ns = info.num_cores, info.num_subcores
    per_tile = total // (nc * ns)
    n_chunks = per_tile // CHUNK
    k = _build(total, per_tile, n_chunks, F, nc, ns)
    out = k(itemEmbedding, r_table.reshape(-1),
            head_ids.reshape(total // CHUNK, CHUNK).astype(jnp.int32),
            tail_ids.reshape(total // CHUNK, CHUNK).astype(jnp.int32),
            relation_ids.reshape(-1).astype(jnp.int32))
    return out.reshape(B, K)


# speedup vs baseline: 1.3363x; 1.2779x over previous
import functools

import jax
import jax.numpy as jnp
from jax import lax
from jax.experimental import pallas as pl
from jax.experimental.pallas import tpu as pltpu
from jax.experimental.pallas import tpu_sc as plsc

L = 16
CHUNK = 128


@functools.lru_cache(maxsize=None)
def _build(total, per_tile, n_chunks, F, nc, ns):
    mesh = plsc.VectorSubcoreMesh(core_axis_name="c", subcore_axis_name="s")
    pv = CHUNK // L

    @functools.partial(
        pl.kernel,
        mesh=mesh,
        compiler_params=pltpu.CompilerParams(
            needs_layout_passes=False, use_tc_tiling_on_sc=True),
        out_type=jax.ShapeDtypeStruct((total,), jnp.float32),
        scratch_types=[
            pltpu.VMEM((per_tile,), jnp.int32),
            pltpu.VMEM((per_tile,), jnp.int32),
            pltpu.VMEM((per_tile,), jnp.int32),
            pltpu.VMEM((CHUNK, F), jnp.float32),
            pltpu.VMEM((CHUNK, F), jnp.float32),
            pltpu.VMEM((CHUNK, F), jnp.float32),
            pltpu.VMEM((CHUNK, F), jnp.float32),
            pltpu.VMEM((2 * F,), jnp.float32),
            pltpu.VMEM((per_tile,), jnp.float32),
            pltpu.SemaphoreType.DMA,
            pltpu.SemaphoreType.DMA,
            pltpu.SemaphoreType.DMA,
            pltpu.SemaphoreType.DMA,
        ],
    )
    def k(table, rtab, hids, tids, rids, out,
          relv, hidv, tidv, hb0, hb1, tb0, tb1,
          rbuf, outbuf, hs0, hs1, ts0, ts1):
        wid = lax.axis_index("s") * nc + lax.axis_index("c")
        base = wid * per_tile
        pltpu.sync_copy(rtab, rbuf)
        pltpu.sync_copy(rids.at[pl.ds(base, per_tile)], relv)
        pltpu.sync_copy(hids.at[pl.ds(base, per_tile)], hidv)
        pltpu.sync_copy(tids.at[pl.ds(base, per_tile)], tidv)
        iota = lax.iota(jnp.int32, L)
        rows = [iota + p * L for p in range(pv)]

        def fire(c, hb, tb):
            off = c * CHUNK
            hsem = hs0 if hb is hb0 else hs1
            tsem = ts0 if tb is tb0 else ts1

            def issue(j, carry):
                g = (j // L) * L
                lane = j % L
                msk = iota == lane
                hvec = hidv[pl.ds(off + g, L)]
                tvec = tidv[pl.ds(off + g, L)]
                hrow = lax.reduce_sum(jnp.where(msk, hvec, 0), axes=(0,))
                trow = lax.reduce_sum(jnp.where(msk, tvec, 0), axes=(0,))
                pltpu.async_copy(
                    table.at[pl.ds(hrow, 1)], hb.at[pl.ds(j, 1)], hsem)
                pltpu.async_copy(
                    table.at[pl.ds(trow, 1)], tb.at[pl.ds(j, 1)], tsem)
                return carry

            lax.fori_loop(0, CHUNK, issue, 0)

        def drain(hb, tb):
            pltpu.make_async_copy(
                table.at[pl.ds(0, CHUNK)], hb,
                hs0 if hb is hb0 else hs1).wait()
            pltpu.make_async_copy(
                table.at[pl.ds(0, CHUNK)], tb,
                ts0 if tb is tb0 else ts1).wait()

        def compute(c, hb, tb):
            rels = [relv[pl.ds(c * CHUNK + p * L, L)] * F for p in range(pv)]

            def body(f, accs):
                fs = jnp.broadcast_to(f.astype(jnp.int32), (L,))
                new = []
                for p in range(pv):
                    hv = plsc.load_gather(hb, [rows[p], fs])
                    tv = plsc.load_gather(tb, [rows[p], fs])
                    rv = plsc.load_gather(rbuf, [rels[p] + fs])
                    d = hv + rv - tv
                    new.append(accs[p] + d * d)
                return tuple(new)

            accs = lax.fori_loop(
                0, F, body,
                tuple(jnp.zeros((L,), jnp.float32) for _ in range(pv)))
            for p in range(pv):
                outbuf[pl.ds(c * CHUNK + p * L, L)] = -accs[p]

        fire(0, hb0, tb0)
        for c in range(n_chunks):
            if c % 2 == 0:
                if c + 1 < n_chunks:
                    fire(c + 1, hb1, tb1)
                drain(hb0, tb0)
                compute(c, hb0, tb0)
            else:
                if c + 1 < n_chunks:
                    fire(c + 1, hb0, tb0)
                drain(hb1, tb1)
                compute(c, hb1, tb1)
        pltpu.sync_copy(outbuf, out.at[pl.ds(base, per_tile)])

    return k


def kernel(itemEmbedding, r_table, head_ids, tail_ids, relation_ids):
    B, K = head_ids.shape
    total = B * K
    F = itemEmbedding.shape[1]
    info = plsc.get_sparse_core_info()
    nc, ns = info.num_cores, info.num_subcores
    per_tile = total // (nc * ns)
    n_chunks = per_tile // CHUNK
    k = _build(total, per_tile, n_chunks, F, nc, ns)
    out = k(itemEmbedding, r_table.reshape(-1),
            head_ids.reshape(-1).astype(jnp.int32),
            tail_ids.reshape(-1).astype(jnp.int32),
            relation_ids.reshape(-1).astype(jnp.int32))
    return out.reshape(B, K)
